# baseline (speedup 1.0000x reference)
import functools

import jax
import jax.numpy as jnp
import numpy as np
from jax import lax
from jax.experimental import pallas as pl
from jax.experimental.pallas import tpu as pltpu
from jax.experimental.pallas import tpu_sc as plsc

_K = 9
_NEG_MIN = np.int32(-2147483648)
_MASK31 = np.int32(0x7FFFFFFF)
_CHUNK = 128
_G = 16


def _gelu(v):
    return jax.nn.gelu(v)


def _mm(a, b):
    return lax.dot_general(a, b, (((1,), (0,)), ((), ())),
                           preferred_element_type=jnp.float32)


def _tc_a_body(x_ref, in1_W1, in1_b1, in1_W2, in1_b2,
               h_out, idx_out, S_ref):
    x = x_ref[0]
    N = x.shape[0]
    h = _mm(_gelu(_mm(x, in1_W1[...]) + in1_b1[...]), in1_W2[...]) + in1_b2[...]
    h_out[0] = h

    sim = lax.dot_general(x, x, (((1,), (1,)), ((), ())),
                          preferred_element_type=jnp.float32)
    b = lax.bitcast_convert_type(sim, jnp.int32)
    s = b ^ (lax.shift_right_arithmetic(b, 31) & _MASK31)
    packed = (s & np.int32(-1024)) | (np.int32(1023) -
                                      lax.broadcasted_iota(jnp.int32, (N, N), 1))
    S_ref[...] = lax.bitcast_convert_type(
        packed ^ (lax.shift_right_arithmetic(packed, 31) & _MASK31),
        jnp.float32)

    fprev = None
    for t in range(_K):
        F = S_ref[...]
        masked = F if t == 0 else jnp.where(F < fprev, F, np.float32(-np.inf))
        m = jnp.max(masked, axis=1, keepdims=True)
        mp = lax.bitcast_convert_type(m, jnp.int32)
        mp = mp ^ (lax.shift_right_arithmetic(mp, 31) & _MASK31)
        idx_out[0, t:t + 1, :] = jnp.reshape(
            np.int32(1023) - (mp & np.int32(1023)), (1, N))
        fprev = m


def _tc_c_body(x_ref, h_ref, maxT_ref, conv_Wh, conv_Wa, conv_b,
               out1_W1, out1_b1, out1_W2, out1_b2,
               in2_W1, in2_b1, in2_W2, in2_b2,
               out2_W1, out2_b1, out2_W2, out2_b2, out_ref):
    x = x_ref[0]
    h = h_ref[0]
    maxT = maxT_ref[0]
    u = (_mm(h, conv_Wh[...]) - _mm(h, conv_Wa[...]) + conv_b[...]
         + lax.dot_general(maxT, conv_Wa[...], (((0,), (0,)), ((), ())),
                           preferred_element_type=jnp.float32))
    g = _gelu(u)
    h2 = _mm(_gelu(_mm(g, out1_W1[...]) + out1_b1[...]), out1_W2[...]) + out1_b2[...]
    hh = h2 + x
    t1 = _mm(_gelu(_mm(hh, in2_W1[...]) + in2_b1[...]), in2_W2[...]) + in2_b2[...]
    t2 = _gelu(t1)
    t3 = _mm(_gelu(_mm(t2, out2_W1[...]) + out2_b1[...]), out2_W2[...]) + out2_b2[...]
    out_ref[0] = t3 + hh


def _make_sc_agg(B, N, C):
    mesh = plsc.VectorSubcoreMesh(core_axis_name="c", subcore_axis_name="s")

    @functools.partial(
        pl.kernel, mesh=mesh,
        compiler_params=pltpu.CompilerParams(needs_layout_passes=False),
        out_type=jax.ShapeDtypeStruct((B, C, N), jnp.float32),
        scratch_types=[
            pltpu.VMEM((N * C,), jnp.float32),
            pltpu.VMEM((_K * N,), jnp.int32),
            pltpu.VMEM((C, _CHUNK), jnp.float32),
        ],
    )
    def sc_agg(h_hbm, idx_hbm, out_hbm, table_v, idx_v, aggT_v):
        w = lax.axis_index("s") * 2 + lax.axis_index("c")
        pltpu.sync_copy(h_hbm.at[w], table_v)
        pltpu.sync_copy(idx_hbm.at[w], idx_v)

        def chunk_body(ck, carry):
            def group_body(g, carry2):
                base = ck * _CHUNK + g * _G
                addrs = [idx_v[pl.ds(t * N + base, _G)] * np.int32(C)
                         for t in range(_K)]
                for c in range(C):
                    acc = plsc.load_gather(table_v, [addrs[0] + np.int32(c)])
                    for t in range(1, _K):
                        acc = jnp.maximum(
                            acc,
                            plsc.load_gather(table_v, [addrs[t] + np.int32(c)]))
                    aggT_v[c, pl.ds(g * _G, _G)] = acc
                return carry2
            lax.fori_loop(0, _CHUNK // _G, group_body, 0)
            pltpu.sync_copy(aggT_v,
                            out_hbm.at[w, :, pl.ds(ck * _CHUNK, _CHUNK)])
            return carry
        lax.fori_loop(0, N // _CHUNK, chunk_body, 0)

    return sc_agg


@jax.jit
def kernel(x, in1_W1, in1_b1, in1_W2, in1_b2, conv_W, conv_b,
           out1_W1, out1_b1, out1_W2, out1_b2,
           in2_W1, in2_b1, in2_W2, in2_b2,
           out2_W1, out2_b1, out2_W2, out2_b2):
    B, N, C = x.shape
    V = conv_W.T
    conv_Wh = V[0::2]
    conv_Wa = V[1::2]

    def row(v):
        return v.reshape(1, -1)

    full = lambda s: pl.BlockSpec(s, lambda b: (0,) * len(s))

    h, idx = pl.pallas_call(
        _tc_a_body,
        grid=(B,),
        in_specs=[pl.BlockSpec((1, N, C), lambda b: (b, 0, 0)),
                  full((C, C)), full((1, C)), full((C, C)), full((1, C))],
        out_specs=[pl.BlockSpec((1, N, C), lambda b: (b, 0, 0)),
                   pl.BlockSpec((1, _K, N), lambda b: (b, 0, 0))],
        out_shape=[jax.ShapeDtypeStruct((B, N, C), jnp.float32),
                   jax.ShapeDtypeStruct((B, _K, N), jnp.int32)],
        scratch_shapes=[pltpu.VMEM((N, N), jnp.float32)],
    )(x, in1_W1, row(in1_b1), in1_W2, row(in1_b2))

    maxT = _make_sc_agg(B, N, C)(h.reshape(B, N * C), idx.reshape(B, _K * N))

    wspecs = [
        full((C, C)), full((C, C)), full((1, C)),
        full((C, C)), full((1, C)), full((C, C)), full((1, C)),
        full((C, 4 * C)), full((1, 4 * C)), full((4 * C, C)), full((1, C)),
        full((C, 4 * C)), full((1, 4 * C)), full((4 * C, C)), full((1, C)),
    ]
    return pl.pallas_call(
        _tc_c_body,
        grid=(B,),
        in_specs=[pl.BlockSpec((1, N, C), lambda b: (b, 0, 0)),
                  pl.BlockSpec((1, N, C), lambda b: (b, 0, 0)),
                  pl.BlockSpec((1, C, N), lambda b: (b, 0, 0))] + wspecs,
        out_specs=pl.BlockSpec((1, N, C), lambda b: (b, 0, 0)),
        out_shape=jax.ShapeDtypeStruct((B, N, C), jnp.float32),
    )(x, h, maxT,
      conv_Wh, conv_Wa, row(conv_b),
      out1_W1, row(out1_b1), out1_W2, row(out1_b2),
      in2_W1, row(in2_b1), in2_W2, row(in2_b2),
      out2_W1, row(out2_b1), out2_W2, row(out2_b2))
